# baseline (speedup 1.0000x reference)
import functools

import jax
import jax.numpy as jnp
from jax import lax
from jax.experimental import pallas as pl
from jax.experimental.pallas import tpu as pltpu
from jax.experimental.pallas import tpu_sc as plsc

_F32 = jnp.float32


def _ceil_to(a, m):
    return -(-a // m) * m


def _fill_const(ref, rows, width, value):
    def body(r, carry):
        for j in range(width // 16):
            ref[r, pl.ds(j * 16, 16)] = jnp.full((16,), value, _F32)
        return carry

    lax.fori_loop(0, rows, body, 0)


def _zero_acc_slice(rows0, acc, sid, RPT, CH, Dk):
    _fill_const(rows0, CH, Dk, 0.0)
    for k in range(-(-RPT // CH)):
        sz = min(CH, RPT - k * CH)
        pltpu.sync_copy(
            rows0.at[pl.ds(0, sz)], acc.at[pl.ds(sid * RPT + k * CH, sz)]
        )


@functools.cache
def _msg_call(NPAD, Dk, EPW, CH):
    n_iter = EPW // CH
    assert n_iter % 2 == 0
    n_outer = n_iter // 2
    RPT = NPAD // 16
    mesh = plsc.VectorSubcoreMesh(core_axis_name="c", subcore_axis_name="s")
    NC = 2

    def body(g_hbm, src_hbm, dst_hbm, out_hbm, idxd, sring, rows, acc,
             gsem0, gsem1, ssem0, ssem1, isem0, isem1):
        cid = lax.axis_index("c")
        sid = lax.axis_index("s")
        wid = sid * NC + cid

        _zero_acc_slice(rows.at[0], acc, sid, RPT, CH, Dk)
        pltpu.sync_copy(dst_hbm.at[wid], idxd)
        pltpu.sync_copy(src_hbm.at[wid, 0], sring.at[0])
        pltpu.sync_copy(src_hbm.at[wid, 1], sring.at[1])
        plsc.subcore_barrier()

        pltpu.async_copy(g_hbm.at[sring.at[0]], rows.at[0], gsem0)

        def outer(o, carry):
            i0 = 2 * o
            i1 = i0 + 1
            pltpu.make_async_copy(
                g_hbm.at[sring.at[0]], rows.at[0], gsem0
            ).wait()

            @pl.when(i0 + 2 < n_iter)
            def _():
                pltpu.async_copy(src_hbm.at[wid, i0 + 2], sring.at[0], isem0)

            pltpu.async_copy(
                rows.at[0], acc.at[idxd.at[i0]], ssem0, add=True
            )

            @pl.when(o > 0)
            def _():
                pltpu.make_async_copy(
                    rows.at[1], acc.at[idxd.at[i0 - 1]], ssem1
                ).wait()
                pltpu.make_async_copy(
                    src_hbm.at[wid, i1], sring.at[1], isem1
                ).wait()

            pltpu.async_copy(g_hbm.at[sring.at[1]], rows.at[1], gsem1)
            pltpu.make_async_copy(
                g_hbm.at[sring.at[1]], rows.at[1], gsem1
            ).wait()

            @pl.when(i1 + 2 < n_iter)
            def _():
                pltpu.async_copy(src_hbm.at[wid, i1 + 2], sring.at[1], isem1)

            pltpu.async_copy(
                rows.at[1], acc.at[idxd.at[i1]], ssem1, add=True
            )
            pltpu.make_async_copy(
                rows.at[0], acc.at[idxd.at[i0]], ssem0
            ).wait()

            @pl.when(i0 + 2 < n_iter)
            def _():
                pltpu.make_async_copy(
                    src_hbm.at[wid, i0 + 2], sring.at[0], isem0
                ).wait()
                pltpu.async_copy(g_hbm.at[sring.at[0]], rows.at[0], gsem0)

            return carry

        lax.fori_loop(0, n_outer, outer, 0)
        pltpu.make_async_copy(
            rows.at[1], acc.at[idxd.at[n_iter - 1]], ssem1
        ).wait()

        plsc.subcore_barrier()
        pltpu.sync_copy(
            acc.at[pl.ds(sid * RPT, RPT)],
            out_hbm.at[pl.ds(cid * NPAD + sid * RPT, RPT)],
        )

    return pl.kernel(
        body,
        out_type=jax.ShapeDtypeStruct((2 * NPAD, Dk), _F32),
        mesh=mesh,
        scratch_types=[
            pltpu.VMEM((n_iter, CH), jnp.int32),
            pltpu.VMEM((2, CH), jnp.int32),
            pltpu.VMEM((2, CH, Dk), _F32),
            pltpu.VMEM_SHARED((NPAD, Dk), _F32),
        ] + [pltpu.SemaphoreType.DMA] * 6,
    )


@functools.cache
def _deg_call(NPAD, EPW, CH):
    n_iter = EPW // CH
    assert n_iter % 2 == 0
    n_outer = n_iter // 2
    RPT = NPAD // 16
    Dk = 16
    mesh = plsc.VectorSubcoreMesh(core_axis_name="c", subcore_axis_name="s")
    NC = 2

    def body(dst_hbm, out_hbm, idxd, rows_v, acc, ssem0, ssem1):
        cid = lax.axis_index("c")
        sid = lax.axis_index("s")
        wid = sid * NC + cid

        _zero_acc_slice(rows_v, acc, sid, RPT, CH, Dk)
        pltpu.sync_copy(dst_hbm.at[wid], idxd)
        plsc.subcore_barrier()
        _fill_const(rows_v, CH, Dk, 1.0)

        def outer(o, carry):
            i0 = 2 * o
            pltpu.async_copy(rows_v, acc.at[idxd.at[i0]], ssem0, add=True)
            pltpu.async_copy(rows_v, acc.at[idxd.at[i0 + 1]], ssem1, add=True)
            pltpu.make_async_copy(rows_v, acc.at[idxd.at[i0]], ssem0).wait()
            pltpu.make_async_copy(rows_v, acc.at[idxd.at[i0 + 1]], ssem1).wait()
            return carry

        lax.fori_loop(0, n_outer, outer, 0)

        plsc.subcore_barrier()
        pltpu.sync_copy(
            acc.at[pl.ds(sid * RPT, RPT)],
            out_hbm.at[pl.ds(cid * NPAD + sid * RPT, RPT)],
        )

    return pl.kernel(
        body,
        out_type=jax.ShapeDtypeStruct((2 * NPAD, Dk), _F32),
        mesh=mesh,
        scratch_types=[
            pltpu.VMEM((n_iter, CH), jnp.int32),
            pltpu.VMEM((CH, Dk), _F32),
            pltpu.VMEM_SHARED((NPAD, Dk), _F32),
        ] + [pltpu.SemaphoreType.DMA] * 2,
    )


@functools.cache
def _tc_a(NPAD, D, H):
    def body(hist_ref, x_ref, w_ref, dinv_ref, g_ref):
        deg = hist_ref[0] + hist_ref[1] + 1.0
        dinv = lax.rsqrt(deg)[:, 0:1]
        h = lax.dot_general(
            x_ref[...], w_ref[...], (((1,), (1,)), ((), ())),
            preferred_element_type=_F32,
        )
        dinv_ref[...] = dinv
        g_ref[...] = h * dinv

    return pl.pallas_call(
        body,
        out_shape=[
            jax.ShapeDtypeStruct((NPAD, 1), _F32),
            jax.ShapeDtypeStruct((NPAD, H), _F32),
        ],
    )


@functools.cache
def _tc_bn(NPAD, N, H, with_matmul):
    def body(p_ref, g_ref, dinv_ref, b_ref, gam_ref, bet_ref, *rest):
        s = (p_ref[0] + p_ref[1] + g_ref[...]) * dinv_ref[...] + b_ref[...]
        pre = s[:N]
        mean = jnp.mean(pre, axis=0, keepdims=True)
        cen = pre - mean
        var = jnp.mean(cen * cen, axis=0, keepdims=True)
        h = jnp.maximum(
            cen * lax.rsqrt(var + 1e-5) * gam_ref[...] + bet_ref[...], 0.0
        )
        if with_matmul:
            w_ref, out_ref = rest
            hp = jnp.concatenate([h, jnp.zeros((NPAD - N, H), _F32)], axis=0)
            out_ref[...] = (
                lax.dot_general(
                    hp, w_ref[...], (((1,), (1,)), ((), ())),
                    preferred_element_type=_F32,
                )
                * dinv_ref[...]
            )
        else:
            (out_ref,) = rest
            out_ref[...] = h

    out_shape = jax.ShapeDtypeStruct((NPAD, H) if with_matmul else (N, H), _F32)
    return pl.pallas_call(body, out_shape=out_shape)


def kernel(x, edge_index, W1, b1, gamma1, beta1, W2, b2, gamma2, beta2):
    N, D = x.shape
    H = W1.shape[0]
    E = edge_index.shape[1]
    NPAD = _ceil_to(N + 1, 128)
    CH = 128
    EPAD = _ceil_to(E, 32 * CH * 2)
    EPW = EPAD // 32

    src = edge_index[0].astype(jnp.int32)
    dst = edge_index[1].astype(jnp.int32)
    if EPAD != E:
        pad = jnp.full((EPAD - E,), N, jnp.int32)
        src = jnp.concatenate([src, pad])
        dst = jnp.concatenate([dst, pad])
    src = src.reshape(32, EPW // CH, CH)
    dst = dst.reshape(32, EPW // CH, CH)
    x_pad = jnp.pad(x, ((0, NPAD - N), (0, 0)))

    hist = _deg_call(NPAD, EPW, CH)(dst).reshape(2, NPAD, 16)
    dinv, g1 = _tc_a(NPAD, D, H)(hist, x_pad, W1)
    p1 = _msg_call(NPAD, H, EPW, CH)(g1, src, dst).reshape(2, NPAD, H)
    g2 = _tc_bn(NPAD, N, H, True)(
        p1, g1, dinv,
        b1.reshape(1, H), gamma1.reshape(1, H), beta1.reshape(1, H), W2,
    )
    p2 = _msg_call(NPAD, H, EPW, CH)(g2, src, dst).reshape(2, NPAD, H)
    out = _tc_bn(NPAD, N, H, False)(
        p2, g2, dinv,
        b2.reshape(1, H), gamma2.reshape(1, H), beta2.reshape(1, H),
    )
    return out


# speedup vs baseline: 1.2272x; 1.2272x over previous
import functools

import jax
import jax.numpy as jnp
from jax import lax
from jax.experimental import pallas as pl
from jax.experimental.pallas import tpu as pltpu
from jax.experimental.pallas import tpu_sc as plsc

_F32 = jnp.float32


def _ceil_to(a, m):
    return -(-a // m) * m


def _fill_const(ref, rows, width, value):
    def body(r, carry):
        for j in range(width // 16):
            ref[r, pl.ds(j * 16, 16)] = jnp.full((16,), value, _F32)
        return carry

    lax.fori_loop(0, rows, body, 0)


def _zero_acc_slice(rows0, acc, sid, RPT, CH, Dk):
    _fill_const(rows0, CH, Dk, 0.0)
    for k in range(-(-RPT // CH)):
        sz = min(CH, RPT - k * CH)
        pltpu.sync_copy(
            rows0.at[pl.ds(0, sz)], acc.at[pl.ds(sid * RPT + k * CH, sz)]
        )


@functools.cache
def _msg_call(NPAD, Dk, EPW, CH):
    n_iter = EPW // CH
    RPT = NPAD // 16
    mesh = plsc.VectorSubcoreMesh(core_axis_name="c", subcore_axis_name="s")
    NC = 2

    def body(g_hbm, src_hbm, dst_hbm, out_hbm, src_v, dst_v, rows_v, acc, sem):
        cid = lax.axis_index("c")
        sid = lax.axis_index("s")
        wid = sid * NC + cid

        _fill_const(rows_v, CH, Dk, 0.0)
        for k in range(-(-RPT // CH)):
            sz = min(CH, RPT - k * CH)
            pltpu.sync_copy(
                rows_v.at[pl.ds(0, sz)], acc.at[pl.ds(sid * RPT + k * CH, sz)]
            )
        plsc.subcore_barrier()

        ebase = wid * EPW

        def step(i, carry):
            base = ebase + i * CH
            pltpu.sync_copy(src_hbm.at[pl.ds(base, CH)], src_v)
            pltpu.sync_copy(dst_hbm.at[pl.ds(base, CH)], dst_v)
            pltpu.async_copy(g_hbm.at[src_v], rows_v, sem).wait()
            pltpu.sync_copy(rows_v, acc.at[dst_v], add=True)
            return carry

        lax.fori_loop(0, n_iter, step, 0)

        plsc.subcore_barrier()
        pltpu.sync_copy(
            acc.at[pl.ds(sid * RPT, RPT)],
            out_hbm.at[pl.ds(cid * NPAD + sid * RPT, RPT)],
        )

    return pl.kernel(
        body,
        out_type=jax.ShapeDtypeStruct((2 * NPAD, Dk), _F32),
        mesh=mesh,
        scratch_types=[
            pltpu.VMEM((CH,), jnp.int32),
            pltpu.VMEM((CH,), jnp.int32),
            pltpu.VMEM((CH, Dk), _F32),
            pltpu.VMEM_SHARED((NPAD, Dk), _F32),
            pltpu.SemaphoreType.DMA,
        ],
    )


@functools.cache
def _deg_call(NPAD, EPW, CH):
    n_iter = EPW // CH
    RPT = NPAD // 16
    Dk = 128
    mesh = plsc.VectorSubcoreMesh(core_axis_name="c", subcore_axis_name="s")
    NC = 2

    def body(dst_hbm, out_hbm, dst_v, rows_v, acc):
        cid = lax.axis_index("c")
        sid = lax.axis_index("s")
        wid = sid * NC + cid

        _fill_const(rows_v, CH, Dk, 0.0)
        for k in range(-(-RPT // CH)):
            sz = min(CH, RPT - k * CH)
            pltpu.sync_copy(
                rows_v.at[pl.ds(0, sz)], acc.at[pl.ds(sid * RPT + k * CH, sz)]
            )
        plsc.subcore_barrier()
        _fill_const(rows_v, CH, Dk, 1.0)

        ebase = wid * EPW

        def step(i, carry):
            base = ebase + i * CH
            pltpu.sync_copy(dst_hbm.at[pl.ds(base, CH)], dst_v)
            pltpu.sync_copy(rows_v, acc.at[dst_v], add=True)
            return carry

        lax.fori_loop(0, n_iter, step, 0)

        plsc.subcore_barrier()
        pltpu.sync_copy(
            acc.at[pl.ds(sid * RPT, RPT)],
            out_hbm.at[pl.ds(cid * NPAD + sid * RPT, RPT)],
        )

    return pl.kernel(
        body,
        out_type=jax.ShapeDtypeStruct((2 * NPAD, Dk), _F32),
        mesh=mesh,
        scratch_types=[
            pltpu.VMEM((CH,), jnp.int32),
            pltpu.VMEM((CH, Dk), _F32),
            pltpu.VMEM_SHARED((NPAD, Dk), _F32),
        ],
    )


@functools.cache
def _tc_a(NPAD, D, H):
    def body(hist_ref, x_ref, w_ref, dinv_ref, g_ref):
        deg = hist_ref[0] + hist_ref[1] + 1.0
        dinv = lax.rsqrt(deg)[:, 0:1]
        h = lax.dot_general(
            x_ref[...], w_ref[...], (((1,), (1,)), ((), ())),
            preferred_element_type=_F32,
        )
        dinv_ref[...] = dinv
        g_ref[...] = h * dinv

    return pl.pallas_call(
        body,
        out_shape=[
            jax.ShapeDtypeStruct((NPAD, 1), _F32),
            jax.ShapeDtypeStruct((NPAD, H), _F32),
        ],
    )


@functools.cache
def _tc_bn(NPAD, N, H, with_matmul):
    def body(p_ref, g_ref, dinv_ref, b_ref, gam_ref, bet_ref, *rest):
        s = (p_ref[0] + p_ref[1] + g_ref[...]) * dinv_ref[...] + b_ref[...]
        pre = s[:N]
        mean = jnp.mean(pre, axis=0, keepdims=True)
        cen = pre - mean
        var = jnp.mean(cen * cen, axis=0, keepdims=True)
        h = jnp.maximum(
            cen * lax.rsqrt(var + 1e-5) * gam_ref[...] + bet_ref[...], 0.0
        )
        if with_matmul:
            w_ref, out_ref = rest
            hp = jnp.concatenate([h, jnp.zeros((NPAD - N, H), _F32)], axis=0)
            out_ref[...] = (
                lax.dot_general(
                    hp, w_ref[...], (((1,), (1,)), ((), ())),
                    preferred_element_type=_F32,
                )
                * dinv_ref[...]
            )
        else:
            (out_ref,) = rest
            out_ref[...] = h

    out_shape = jax.ShapeDtypeStruct((NPAD, H) if with_matmul else (N, H), _F32)
    return pl.pallas_call(body, out_shape=out_shape)


def kernel(x, edge_index, W1, b1, gamma1, beta1, W2, b2, gamma2, beta2):
    N, D = x.shape
    H = W1.shape[0]
    E = edge_index.shape[1]
    NPAD = _ceil_to(N + 1, 128)
    CH = 128
    EPAD = _ceil_to(E, 32 * CH)
    EPW = EPAD // 32

    src = edge_index[0].astype(jnp.int32)
    dst = edge_index[1].astype(jnp.int32)
    if EPAD != E:
        pad = jnp.full((EPAD - E,), N, jnp.int32)
        src = jnp.concatenate([src, pad])
        dst = jnp.concatenate([dst, pad])
    x_pad = jnp.pad(x, ((0, NPAD - N), (0, 0)))

    hist = _deg_call(NPAD, EPW, CH)(dst).reshape(2, NPAD, H)
    dinv, g1 = _tc_a(NPAD, D, H)(hist, x_pad, W1)
    p1 = _msg_call(NPAD, H, EPW, CH)(g1, src, dst).reshape(2, NPAD, H)
    g2 = _tc_bn(NPAD, N, H, True)(
        p1, g1, dinv,
        b1.reshape(1, H), gamma1.reshape(1, H), beta1.reshape(1, H), W2,
    )
    p2 = _msg_call(NPAD, H, EPW, CH)(g2, src, dst).reshape(2, NPAD, H)
    out = _tc_bn(NPAD, N, H, False)(
        p2, g2, dinv,
        b2.reshape(1, H), gamma2.reshape(1, H), beta2.reshape(1, H),
    )
    return out


# speedup vs baseline: 1.3898x; 1.1325x over previous
import functools

import jax
import jax.numpy as jnp
from jax import lax
from jax.experimental import pallas as pl
from jax.experimental.pallas import tpu as pltpu
from jax.experimental.pallas import tpu_sc as plsc

_F32 = jnp.float32


def _ceil_to(a, m):
    return -(-a // m) * m


def _fill_const(ref, rows, width, value):
    def body(r, carry):
        for j in range(width // 16):
            ref[r, pl.ds(j * 16, 16)] = jnp.full((16,), value, _F32)
        return carry

    lax.fori_loop(0, rows, body, 0)


def _zero_acc_slice(rows0, acc, sid, RPT, CH, Dk):
    _fill_const(rows0, CH, Dk, 0.0)
    for k in range(-(-RPT // CH)):
        sz = min(CH, RPT - k * CH)
        pltpu.sync_copy(
            rows0.at[pl.ds(0, sz)], acc.at[pl.ds(sid * RPT + k * CH, sz)]
        )


@functools.cache
def _msg_call(NPAD, Dk, EPW, CH):
    n_iter = EPW // CH
    RPT = NPAD // 16
    mesh = plsc.VectorSubcoreMesh(core_axis_name="c", subcore_axis_name="s")
    NC = 2

    def body(g_hbm, src_hbm, dst_hbm, out_hbm, idxs, idxd, rows, acc, gsem):
        cid = lax.axis_index("c")
        sid = lax.axis_index("s")
        wid = sid * NC + cid

        _zero_acc_slice(rows, acc, sid, RPT, CH, Dk)
        pltpu.sync_copy(src_hbm.at[wid], idxs)
        pltpu.sync_copy(dst_hbm.at[wid], idxd)
        plsc.subcore_barrier()

        def step(i, carry):
            pltpu.async_copy(g_hbm.at[idxs.at[i]], rows, gsem).wait()
            pltpu.sync_copy(rows, acc.at[idxd.at[i]], add=True)
            return carry

        lax.fori_loop(0, n_iter, step, 0)

        plsc.subcore_barrier()
        pltpu.sync_copy(
            acc.at[pl.ds(sid * RPT, RPT)],
            out_hbm.at[pl.ds(cid * NPAD + sid * RPT, RPT)],
        )

    return pl.kernel(
        body,
        out_type=jax.ShapeDtypeStruct((2 * NPAD, Dk), _F32),
        mesh=mesh,
        scratch_types=[
            pltpu.VMEM((n_iter, CH), jnp.int32),
            pltpu.VMEM((n_iter, CH), jnp.int32),
            pltpu.VMEM((CH, Dk), _F32),
            pltpu.VMEM_SHARED((NPAD, Dk), _F32),
            pltpu.SemaphoreType.DMA,
        ],
    )


@functools.cache
def _deg_call(NPAD, EPW, CH):
    n_iter = EPW // CH
    RPT = NPAD // 16
    Dk = 128
    mesh = plsc.VectorSubcoreMesh(core_axis_name="c", subcore_axis_name="s")
    NC = 2

    def body(dst_hbm, out_hbm, dst_v, rows_v, acc):
        cid = lax.axis_index("c")
        sid = lax.axis_index("s")
        wid = sid * NC + cid

        _fill_const(rows_v, CH, Dk, 0.0)
        for k in range(-(-RPT // CH)):
            sz = min(CH, RPT - k * CH)
            pltpu.sync_copy(
                rows_v.at[pl.ds(0, sz)], acc.at[pl.ds(sid * RPT + k * CH, sz)]
            )
        plsc.subcore_barrier()
        _fill_const(rows_v, CH, Dk, 1.0)

        ebase = wid * EPW

        def step(i, carry):
            base = ebase + i * CH
            pltpu.sync_copy(dst_hbm.at[pl.ds(base, CH)], dst_v)
            pltpu.sync_copy(rows_v, acc.at[dst_v], add=True)
            return carry

        lax.fori_loop(0, n_iter, step, 0)

        plsc.subcore_barrier()
        pltpu.sync_copy(
            acc.at[pl.ds(sid * RPT, RPT)],
            out_hbm.at[pl.ds(cid * NPAD + sid * RPT, RPT)],
        )

    return pl.kernel(
        body,
        out_type=jax.ShapeDtypeStruct((2 * NPAD, Dk), _F32),
        mesh=mesh,
        scratch_types=[
            pltpu.VMEM((CH,), jnp.int32),
            pltpu.VMEM((CH, Dk), _F32),
            pltpu.VMEM_SHARED((NPAD, Dk), _F32),
        ],
    )


@functools.cache
def _tc_a(NPAD, D, H):
    def body(hist_ref, x_ref, w_ref, dinv_ref, g_ref):
        deg = hist_ref[0] + hist_ref[1] + 1.0
        dinv = lax.rsqrt(deg)[:, 0:1]
        h = lax.dot_general(
            x_ref[...], w_ref[...], (((1,), (1,)), ((), ())),
            preferred_element_type=_F32,
        )
        dinv_ref[...] = dinv
        g_ref[...] = h * dinv

    return pl.pallas_call(
        body,
        out_shape=[
            jax.ShapeDtypeStruct((NPAD, 1), _F32),
            jax.ShapeDtypeStruct((NPAD, H), _F32),
        ],
    )


@functools.cache
def _tc_bn(NPAD, N, H, with_matmul):
    def body(p_ref, g_ref, dinv_ref, b_ref, gam_ref, bet_ref, *rest):
        s = (p_ref[0] + p_ref[1] + g_ref[...]) * dinv_ref[...] + b_ref[...]
        pre = s[:N]
        mean = jnp.mean(pre, axis=0, keepdims=True)
        cen = pre - mean
        var = jnp.mean(cen * cen, axis=0, keepdims=True)
        h = jnp.maximum(
            cen * lax.rsqrt(var + 1e-5) * gam_ref[...] + bet_ref[...], 0.0
        )
        if with_matmul:
            w_ref, out_ref = rest
            hp = jnp.concatenate([h, jnp.zeros((NPAD - N, H), _F32)], axis=0)
            out_ref[...] = (
                lax.dot_general(
                    hp, w_ref[...], (((1,), (1,)), ((), ())),
                    preferred_element_type=_F32,
                )
                * dinv_ref[...]
            )
        else:
            (out_ref,) = rest
            out_ref[...] = h

    out_shape = jax.ShapeDtypeStruct((NPAD, H) if with_matmul else (N, H), _F32)
    return pl.pallas_call(body, out_shape=out_shape)


def kernel(x, edge_index, W1, b1, gamma1, beta1, W2, b2, gamma2, beta2):
    N, D = x.shape
    H = W1.shape[0]
    E = edge_index.shape[1]
    NPAD = _ceil_to(N + 1, 128)
    CH = 128
    EPAD = _ceil_to(E, 32 * CH)
    EPW = EPAD // 32

    src = edge_index[0].astype(jnp.int32)
    dst = edge_index[1].astype(jnp.int32)
    if EPAD != E:
        pad = jnp.full((EPAD - E,), N, jnp.int32)
        src = jnp.concatenate([src, pad])
        dst = jnp.concatenate([dst, pad])
    src3 = src.reshape(32, EPW // CH, CH)
    dst3 = dst.reshape(32, EPW // CH, CH)
    x_pad = jnp.pad(x, ((0, NPAD - N), (0, 0)))

    hist = _deg_call(NPAD, EPW, CH)(dst).reshape(2, NPAD, H)
    dinv, g1 = _tc_a(NPAD, D, H)(hist, x_pad, W1)
    p1 = _msg_call(NPAD, H, EPW, CH)(g1, src3, dst3).reshape(2, NPAD, H)
    g2 = _tc_bn(NPAD, N, H, True)(
        p1, g1, dinv,
        b1.reshape(1, H), gamma1.reshape(1, H), beta1.reshape(1, H), W2,
    )
    p2 = _msg_call(NPAD, H, EPW, CH)(g2, src3, dst3).reshape(2, NPAD, H)
    out = _tc_bn(NPAD, N, H, False)(
        p2, g2, dinv,
        b2.reshape(1, H), gamma2.reshape(1, H), beta2.reshape(1, H),
    )
    return out
